# Initial kernel scaffold; baseline (speedup 1.0000x reference)
import jax
import jax.numpy as jnp
from jax.experimental import pallas as pl


def kernel(x, edge_index, W, b):
    raise NotImplementedError("write your pallas kernel here")


# speedup vs baseline: 23.1036x; 23.1036x over previous
import functools

import jax
import jax.numpy as jnp
from jax import lax
from jax.experimental import pallas as pl
from jax.experimental.pallas import tpu as pltpu
from jax.experimental.pallas import tpu_sc as plsc

N_NODES = 10000
D_FEAT = 128
D_HID = 128

NC = 2
NS = 16
CH = 128
NPAD = 10240
ROWS_PER_TILE = NPAD // NS

NPASS = 4
NPP = NPAD // NPASS
HR = NPP * 16 // 128
DR = NPASS * HR

_MESH = dict(core_axis_name="c", subcore_axis_name="s")


def _deg_kernel_factory(K):
    HW = HR * 128

    @functools.partial(
        pl.kernel,
        out_type=jax.ShapeDtypeStruct((NC * NS * NPASS, HW), jnp.float32),
        mesh=plsc.VectorSubcoreMesh(**_MESH),
        compiler_params=pltpu.CompilerParams(needs_layout_passes=False),
        scratch_types=[
            pltpu.VMEM((K * 8, 16), jnp.int32),
            pltpu.VMEM((HW + 128,), jnp.float32),
        ],
    )
    def deg_kernel(dstp16_hbm, zeros_hbm, out_hbm, idx_v, hist_v):
        c = lax.axis_index("c")
        s = lax.axis_index("s")
        w = c * NS + s
        pltpu.sync_copy(dstp16_hbm.at[pl.ds(w * K * 8, K * 8)], idx_v)
        lane = lax.iota(jnp.int32, 16)
        ones16 = jnp.ones((16,), jnp.float32)
        for p in range(NPASS):
            lo = p * NPP
            pltpu.sync_copy(zeros_hbm, hist_v)

            def body(g, carry):
                d = idx_v[g]
                m = (d >= lo) & (d < lo + NPP)
                flat = jnp.where(m, (d - lo) * 16 + lane, HW + lane)
                plsc.addupdate_scatter(hist_v, [flat], ones16)
                return carry

            lax.fori_loop(0, K * 8, body, 0)
            pltpu.sync_copy(hist_v.at[pl.ds(0, HW)], out_hbm.at[w * NPASS + p])

    return deg_kernel


def _msg_kernel_factory(K):
    @functools.partial(
        pl.kernel,
        out_type=jax.ShapeDtypeStruct((NC * NPAD, D_HID), jnp.float32),
        mesh=plsc.VectorSubcoreMesh(**_MESH),
        scratch_types=[
            pltpu.VMEM((K, CH), jnp.int32),
            pltpu.VMEM((K, CH), jnp.int32),
            pltpu.VMEM((CH, D_HID), jnp.float32),
            pltpu.VMEM_SHARED((NPAD, D_HID), jnp.float32),
            pltpu.SemaphoreType.DMA,
        ],
    )
    def msg_kernel(hs_hbm, srcp_hbm, dstp_hbm, out_hbm,
                   idxs_v, idxd_v, rows_v, acc_sh, sem):
        c = lax.axis_index("c")
        s = lax.axis_index("s")
        r0 = s * ROWS_PER_TILE
        pltpu.sync_copy(hs_hbm.at[pl.ds(r0, ROWS_PER_TILE)],
                        acc_sh.at[pl.ds(r0, ROWS_PER_TILE)])
        base = (c * NS + s) * K
        pltpu.sync_copy(srcp_hbm.at[pl.ds(base, K)], idxs_v)
        pltpu.sync_copy(dstp_hbm.at[pl.ds(base, K)], idxd_v)
        plsc.subcore_barrier()

        def body(j, carry):
            pltpu.async_copy(hs_hbm.at[idxs_v.at[j]], rows_v, sem).wait()
            pltpu.sync_copy(rows_v, acc_sh.at[idxd_v.at[j]], add=True)
            return carry

        lax.fori_loop(0, K, body, 0)
        plsc.subcore_barrier()
        pltpu.sync_copy(acc_sh.at[pl.ds(r0, ROWS_PER_TILE)],
                        out_hbm.at[pl.ds(c * NPAD + r0, ROWS_PER_TILE)])

    return msg_kernel


def _spectral_norm_body(w_ref, out_ref):
    W = w_ref[...]
    din = W.shape[0]
    u = jnp.full((1, din), 1.0 / jnp.sqrt(jnp.float32(din)), dtype=jnp.float32)
    v = u
    for _ in range(3):
        v = lax.dot_general(u, W, (((1,), (0,)), ((), ())), precision=lax.Precision.HIGHEST)
        v = v / (jnp.sqrt(jnp.sum(v * v)) + 1e-12)
        u = lax.dot_general(v, W, (((1,), (1,)), ((), ())), precision=lax.Precision.HIGHEST)
        u = u / (jnp.sqrt(jnp.sum(u * u)) + 1e-12)
    wv = lax.dot_general(v, W, (((1,), (1,)), ((), ())), precision=lax.Precision.HIGHEST)
    sigma = jnp.sum(u * wv)
    out_ref[...] = W / sigma


def _hs_body(x_ref, w_ref, deg_ref, hs_ref, disb_ref):
    s1 = jnp.sum(deg_ref[...], axis=(0, 1))
    r8 = s1.shape[0]
    m_fold = (lax.broadcasted_iota(jnp.int32, (128, 8), 0) // 16
              == lax.broadcasted_iota(jnp.int32, (128, 8), 1)).astype(jnp.float32)
    deg = jnp.dot(s1, m_fold, preferred_element_type=jnp.float32, precision=lax.Precision.HIGHEST)
    dis = lax.rsqrt(deg + 1.0)
    m_exp = (lax.broadcasted_iota(jnp.int32, (8, 8 * 128), 1) // 128
             == lax.broadcasted_iota(jnp.int32, (8, 8 * 128), 0)).astype(jnp.float32)
    dis_b = jnp.dot(dis, m_exp, preferred_element_type=jnp.float32, precision=lax.Precision.HIGHEST)
    dis_b = dis_b.reshape(r8 * 8, 128)
    h = jnp.dot(x_ref[...], w_ref[...], preferred_element_type=jnp.float32, precision=lax.Precision.HIGHEST)
    hs_ref[...] = h * dis_b
    disb_ref[...] = dis_b


def _final_body(a_ref, hs_ref, disb_ref, b_ref, o_ref):
    acc = a_ref[0] + a_ref[1] - hs_ref[...]
    o_ref[...] = acc * disb_ref[...] + b_ref[...]


def kernel(x, edge_index, W, b):
    E = edge_index.shape[1]
    src = edge_index[0].astype(jnp.int32)
    dst = edge_index[1].astype(jnp.int32)

    edges_per_blk = NC * NS * CH * 8
    EP = ((E + edges_per_blk - 1) // edges_per_blk) * edges_per_blk
    K = EP // (NC * NS * CH)
    npad_e = EP - E
    if npad_e:
        pad_rows = N_NODES + (jnp.arange(npad_e, dtype=jnp.int32) % (NPAD - N_NODES))
        srcp = jnp.concatenate([src, pad_rows])
        dstp = jnp.concatenate([dst, pad_rows])
    else:
        srcp, dstp = src, dst
    srcp = srcp.reshape(NC * NS * K, CH)
    dstp = dstp.reshape(NC * NS * K, CH)

    x_pad = jnp.pad(x, ((0, NPAD - N_NODES), (0, 0)))
    zeros_in = jnp.zeros((HR * 128 + 128,), jnp.float32)

    deg_parts = _deg_kernel_factory(K)(dstp.reshape(-1, 16), zeros_in)
    deg_parts = deg_parts.reshape(NC * NS, NPASS, HR, 128)

    w_sn = pl.pallas_call(
        _spectral_norm_body,
        out_shape=jax.ShapeDtypeStruct(W.shape, jnp.float32),
    )(W)

    grid = NPAD // ROWS_PER_TILE
    blk8 = ROWS_PER_TILE // 8
    bpp = NPP // ROWS_PER_TILE
    hs, dis_b = pl.pallas_call(
        _hs_body,
        grid=(grid,),
        in_specs=[
            pl.BlockSpec((ROWS_PER_TILE, D_FEAT), lambda i: (i, 0)),
            pl.BlockSpec((D_FEAT, D_HID), lambda i: (0, 0)),
            pl.BlockSpec((NC * NS, 1, blk8, 128),
                         lambda i: (0, i // bpp, i % bpp, 0)),
        ],
        out_specs=[
            pl.BlockSpec((ROWS_PER_TILE, D_HID), lambda i: (i, 0)),
            pl.BlockSpec((ROWS_PER_TILE, 128), lambda i: (i, 0)),
        ],
        out_shape=[
            jax.ShapeDtypeStruct((NPAD, D_HID), jnp.float32),
            jax.ShapeDtypeStruct((NPAD, 128), jnp.float32),
        ],
    )(x_pad, w_sn, deg_parts)

    acc = _msg_kernel_factory(K)(hs, srcp, dstp)
    acc = acc.reshape(NC, NPAD, D_HID)

    out = pl.pallas_call(
        _final_body,
        grid=(grid,),
        in_specs=[
            pl.BlockSpec((NC, ROWS_PER_TILE, D_HID), lambda i: (0, i, 0)),
            pl.BlockSpec((ROWS_PER_TILE, D_HID), lambda i: (i, 0)),
            pl.BlockSpec((ROWS_PER_TILE, 128), lambda i: (i, 0)),
            pl.BlockSpec((1, D_HID), lambda i: (0, 0)),
        ],
        out_specs=pl.BlockSpec((ROWS_PER_TILE, D_HID), lambda i: (i, 0)),
        out_shape=jax.ShapeDtypeStruct((NPAD, D_HID), jnp.float32),
    )(acc, hs, dis_b, b[None, :])

    return out[:N_NODES]


# speedup vs baseline: 29.1249x; 1.2606x over previous
import functools

import jax
import jax.numpy as jnp
from jax import lax
from jax.experimental import pallas as pl
from jax.experimental.pallas import tpu as pltpu
from jax.experimental.pallas import tpu_sc as plsc

N_NODES = 10000
D_FEAT = 128
D_HID = 128

NC = 2
NS = 16
CH = 128
NPAD = 10240
ROWS_PER_TILE = NPAD // NS

NPASS = 4
NPP = NPAD // NPASS
HR = NPP * 16 // 128
DR = NPASS * HR

_MESH = dict(core_axis_name="c", subcore_axis_name="s")


def _deg_kernel_factory(K):
    HW = HR * 128

    @functools.partial(
        pl.kernel,
        out_type=jax.ShapeDtypeStruct((NC * NS * NPASS, HW), jnp.float32),
        mesh=plsc.VectorSubcoreMesh(**_MESH),
        compiler_params=pltpu.CompilerParams(needs_layout_passes=False),
        scratch_types=[
            pltpu.VMEM((K * 8, 16), jnp.int32),
            pltpu.VMEM((HW + 128,), jnp.float32),
        ],
    )
    def deg_kernel(dstp16_hbm, zeros_hbm, out_hbm, idx_v, hist_v):
        c = lax.axis_index("c")
        s = lax.axis_index("s")
        w = c * NS + s
        pltpu.sync_copy(dstp16_hbm.at[pl.ds(w * K * 8, K * 8)], idx_v)
        lane = lax.iota(jnp.int32, 16)
        ones16 = jnp.ones((16,), jnp.float32)
        for p in range(NPASS):
            lo = p * NPP
            pltpu.sync_copy(zeros_hbm, hist_v)

            def body(g, carry):
                d = idx_v[g]
                m = (d >= lo) & (d < lo + NPP)
                flat = jnp.where(m, (d - lo) * 16 + lane, HW + lane)
                plsc.addupdate_scatter(hist_v, [flat], ones16)
                return carry

            lax.fori_loop(0, K * 8, body, 0)
            pltpu.sync_copy(hist_v.at[pl.ds(0, HW)], out_hbm.at[w * NPASS + p])

    return deg_kernel


def _msg_kernel_factory(K):
    assert K % 2 == 0

    @functools.partial(
        pl.kernel,
        out_type=jax.ShapeDtypeStruct((NC * NPAD, D_HID), jnp.float32),
        mesh=plsc.VectorSubcoreMesh(**_MESH),
        scratch_types=[
            pltpu.VMEM((K, CH), jnp.int32),
            pltpu.VMEM((2, CH), jnp.int32),
            pltpu.VMEM((2, CH), jnp.int32),
            pltpu.VMEM((2, CH, D_HID), jnp.float32),
            pltpu.VMEM_SHARED((NPAD, D_HID), jnp.float32),
            pltpu.SemaphoreType.DMA,
            pltpu.SemaphoreType.DMA,
        ],
    )
    def msg_kernel(hs_hbm, comb_hbm, out_hbm,
                   comb_v, idxs_v, idxd_v, rows_v, acc_sh, sem0, sem1):
        c = lax.axis_index("c")
        s = lax.axis_index("s")
        r0 = s * ROWS_PER_TILE
        pltpu.sync_copy(hs_hbm.at[pl.ds(r0, ROWS_PER_TILE)],
                        acc_sh.at[pl.ds(r0, ROWS_PER_TILE)])
        base = (c * NS + s) * K
        pltpu.sync_copy(comb_hbm.at[pl.ds(base, K)], comb_v)
        plsc.subcore_barrier()

        def unpack(j, p):
            for i in range(CH // 16):
                comb = comb_v[j, pl.ds(i * 16, 16)]
                idxs_v[p, pl.ds(i * 16, 16)] = comb & 0xFFFF
                idxd_v[p, pl.ds(i * 16, 16)] = comb >> 16

        def gather(j, p, sem):
            return pltpu.async_copy(hs_hbm.at[idxs_v.at[p]], rows_v.at[p], sem)

        def scatter(p):
            pltpu.sync_copy(rows_v.at[p], acc_sh.at[idxd_v.at[p]], add=True)

        unpack(0, 0)
        g0 = gather(0, 0, sem0)

        def body(jj, carry):
            j0 = jj * 2
            unpack(j0 + 1, 1)
            g1 = gather(j0 + 1, 1, sem1)
            g0 = pltpu.make_async_copy(hs_hbm.at[idxs_v.at[0]], rows_v.at[0], sem0)
            g0.wait()
            scatter(0)
            unpack(j0 + 2, 0)
            gather(j0 + 2, 0, sem0)
            g1.wait()
            scatter(1)
            return carry

        lax.fori_loop(0, K // 2 - 1, body, 0)
        unpack(K - 1, 1)
        g1 = gather(K - 1, 1, sem1)
        pltpu.make_async_copy(hs_hbm.at[idxs_v.at[0]], rows_v.at[0], sem0).wait()
        scatter(0)
        g1.wait()
        scatter(1)

        plsc.subcore_barrier()
        pltpu.sync_copy(acc_sh.at[pl.ds(r0, ROWS_PER_TILE)],
                        out_hbm.at[pl.ds(c * NPAD + r0, ROWS_PER_TILE)])

    return msg_kernel


def _spectral_norm_body(w_ref, out_ref):
    W = w_ref[...]
    din = W.shape[0]
    u = jnp.full((1, din), 1.0 / jnp.sqrt(jnp.float32(din)), dtype=jnp.float32)
    v = u
    for _ in range(3):
        v = lax.dot_general(u, W, (((1,), (0,)), ((), ())), precision=lax.Precision.HIGHEST)
        v = v / (jnp.sqrt(jnp.sum(v * v)) + 1e-12)
        u = lax.dot_general(v, W, (((1,), (1,)), ((), ())), precision=lax.Precision.HIGHEST)
        u = u / (jnp.sqrt(jnp.sum(u * u)) + 1e-12)
    wv = lax.dot_general(v, W, (((1,), (1,)), ((), ())), precision=lax.Precision.HIGHEST)
    sigma = jnp.sum(u * wv)
    out_ref[...] = W / sigma


def _hs_body(x_ref, w_ref, deg_ref, hs_ref, disb_ref):
    s1 = jnp.sum(deg_ref[...], axis=(0, 1))
    r8 = s1.shape[0]
    m_fold = (lax.broadcasted_iota(jnp.int32, (128, 8), 0) // 16
              == lax.broadcasted_iota(jnp.int32, (128, 8), 1)).astype(jnp.float32)
    deg = jnp.dot(s1, m_fold, preferred_element_type=jnp.float32, precision=lax.Precision.HIGHEST)
    dis = lax.rsqrt(deg + 1.0)
    m_exp = (lax.broadcasted_iota(jnp.int32, (8, 8 * 128), 1) // 128
             == lax.broadcasted_iota(jnp.int32, (8, 8 * 128), 0)).astype(jnp.float32)
    dis_b = jnp.dot(dis, m_exp, preferred_element_type=jnp.float32, precision=lax.Precision.HIGHEST)
    dis_b = dis_b.reshape(r8 * 8, 128)
    h = jnp.dot(x_ref[...], w_ref[...], preferred_element_type=jnp.float32, precision=lax.Precision.HIGHEST)
    hs_ref[...] = h * dis_b
    disb_ref[...] = dis_b


def _final_body(a_ref, hs_ref, disb_ref, b_ref, o_ref):
    acc = a_ref[0] + a_ref[1] - hs_ref[...]
    o_ref[...] = acc * disb_ref[...] + b_ref[...]


def kernel(x, edge_index, W, b):
    E = edge_index.shape[1]
    src = edge_index[0].astype(jnp.int32)
    dst = edge_index[1].astype(jnp.int32)

    edges_per_blk = NC * NS * CH * 8
    EP = ((E + edges_per_blk - 1) // edges_per_blk) * edges_per_blk
    K = EP // (NC * NS * CH)
    npad_e = EP - E
    if npad_e:
        pad_rows = N_NODES + (jnp.arange(npad_e, dtype=jnp.int32) % (NPAD - N_NODES))
        srcp = jnp.concatenate([src, pad_rows])
        dstp = jnp.concatenate([dst, pad_rows])
    else:
        srcp, dstp = src, dst
    srcp = srcp.reshape(NC * NS * K, CH)
    dstp = dstp.reshape(NC * NS * K, CH)

    x_pad = jnp.pad(x, ((0, NPAD - N_NODES), (0, 0)))
    zeros_in = jnp.zeros((HR * 128 + 128,), jnp.float32)

    deg_parts = _deg_kernel_factory(K)(dstp.reshape(-1, 16), zeros_in)
    deg_parts = deg_parts.reshape(NC * NS, NPASS, HR, 128)

    w_sn = pl.pallas_call(
        _spectral_norm_body,
        out_shape=jax.ShapeDtypeStruct(W.shape, jnp.float32),
    )(W)

    grid = NPAD // ROWS_PER_TILE
    blk8 = ROWS_PER_TILE // 8
    bpp = NPP // ROWS_PER_TILE
    hs, dis_b = pl.pallas_call(
        _hs_body,
        grid=(grid,),
        in_specs=[
            pl.BlockSpec((ROWS_PER_TILE, D_FEAT), lambda i: (i, 0)),
            pl.BlockSpec((D_FEAT, D_HID), lambda i: (0, 0)),
            pl.BlockSpec((NC * NS, 1, blk8, 128),
                         lambda i: (0, i // bpp, i % bpp, 0)),
        ],
        out_specs=[
            pl.BlockSpec((ROWS_PER_TILE, D_HID), lambda i: (i, 0)),
            pl.BlockSpec((ROWS_PER_TILE, 128), lambda i: (i, 0)),
        ],
        out_shape=[
            jax.ShapeDtypeStruct((NPAD, D_HID), jnp.float32),
            jax.ShapeDtypeStruct((NPAD, 128), jnp.float32),
        ],
    )(x_pad, w_sn, deg_parts)

    comb = srcp | (dstp << 16)
    acc = _msg_kernel_factory(K)(hs, comb)
    acc = acc.reshape(NC, NPAD, D_HID)

    out = pl.pallas_call(
        _final_body,
        grid=(grid,),
        in_specs=[
            pl.BlockSpec((NC, ROWS_PER_TILE, D_HID), lambda i: (0, i, 0)),
            pl.BlockSpec((ROWS_PER_TILE, D_HID), lambda i: (i, 0)),
            pl.BlockSpec((ROWS_PER_TILE, 128), lambda i: (i, 0)),
            pl.BlockSpec((1, D_HID), lambda i: (0, 0)),
        ],
        out_specs=pl.BlockSpec((ROWS_PER_TILE, D_HID), lambda i: (i, 0)),
        out_shape=jax.ShapeDtypeStruct((NPAD, D_HID), jnp.float32),
    )(acc, hs, dis_b, b[None, :])

    return out[:N_NODES]


# speedup vs baseline: 34.7095x; 1.1917x over previous
import functools

import jax
import jax.numpy as jnp
from jax import lax
from jax.experimental import pallas as pl
from jax.experimental.pallas import tpu as pltpu
from jax.experimental.pallas import tpu_sc as plsc

N_NODES = 10000
D_FEAT = 128
D_HID = 128

NC = 2
NS = 16
CH = 128
NPAD = 10240
ROWS_PER_TILE = NPAD // NS

NPASS = 4
NPP = NPAD // NPASS
HR = NPP * 16 // 128
DR = NPASS * HR

_MESH = dict(core_axis_name="c", subcore_axis_name="s")


def _deg_kernel_factory(K):
    @functools.partial(
        pl.kernel,
        out_type=jax.ShapeDtypeStruct((NC * NS, NPAD), jnp.float32),
        mesh=plsc.VectorSubcoreMesh(**_MESH),
        compiler_params=pltpu.CompilerParams(needs_layout_passes=False),
        scratch_types=[
            pltpu.VMEM((K * 8, 16), jnp.int32),
            pltpu.VMEM((16 * NPP + 16,), jnp.float32),
            pltpu.VMEM((NPP,), jnp.float32),
        ],
    )
    def deg_kernel(dstp16_hbm, zeros_hbm, out_hbm, idx_v, hist_v, cnt_v):
        c = lax.axis_index("c")
        s = lax.axis_index("s")
        w = c * NS + s
        pltpu.sync_copy(dstp16_hbm.at[pl.ds(w * K * 8, K * 8)], idx_v)
        pltpu.sync_copy(zeros_hbm, hist_v)
        lane = lax.iota(jnp.int32, 16)
        ones16 = jnp.ones((16,), jnp.float32)
        zero16 = jnp.zeros((16,), jnp.float32)
        for p in range(NPASS):
            lo = p * NPP

            def body(g, carry):
                d = idx_v[g]
                m = (d >= lo) & (d < lo + NPP)
                flat = jnp.where(m, lane * NPP + (d - lo), 16 * NPP + lane)
                plsc.addupdate_scatter(hist_v, [flat], ones16)
                return carry

            lax.fori_loop(0, K * 8, body, 0, unroll=4)

            def fold(i, carry):
                acc = zero16
                for l in range(16):
                    acc = acc + hist_v[pl.ds(l * NPP + i * 16, 16)]
                    hist_v[pl.ds(l * NPP + i * 16, 16)] = zero16
                cnt_v[pl.ds(i * 16, 16)] = acc
                return carry

            lax.fori_loop(0, NPP // 16, fold, 0, unroll=2)
            pltpu.sync_copy(cnt_v, out_hbm.at[w, pl.ds(lo, NPP)])

    return deg_kernel


def _msg_kernel_factory(K):
    assert K % 2 == 0

    @functools.partial(
        pl.kernel,
        out_type=jax.ShapeDtypeStruct((NC * NPAD, D_HID), jnp.float32),
        mesh=plsc.VectorSubcoreMesh(**_MESH),
        scratch_types=[
            pltpu.VMEM((K, CH), jnp.int32),
            pltpu.VMEM((2, CH), jnp.int32),
            pltpu.VMEM((2, CH), jnp.int32),
            pltpu.VMEM((2, CH, D_HID), jnp.float32),
            pltpu.VMEM_SHARED((NPAD, D_HID), jnp.float32),
            pltpu.SemaphoreType.DMA,
            pltpu.SemaphoreType.DMA,
        ],
    )
    def msg_kernel(hs_hbm, comb_hbm, out_hbm,
                   comb_v, idxs_v, idxd_v, rows_v, acc_sh, sem0, sem1):
        c = lax.axis_index("c")
        s = lax.axis_index("s")
        r0 = s * ROWS_PER_TILE
        pltpu.sync_copy(hs_hbm.at[pl.ds(r0, ROWS_PER_TILE)],
                        acc_sh.at[pl.ds(r0, ROWS_PER_TILE)])
        base = (c * NS + s) * K
        pltpu.sync_copy(comb_hbm.at[pl.ds(base, K)], comb_v)
        plsc.subcore_barrier()

        def unpack(j, p):
            for i in range(CH // 16):
                comb = comb_v[j, pl.ds(i * 16, 16)]
                idxs_v[p, pl.ds(i * 16, 16)] = comb & 0xFFFF
                idxd_v[p, pl.ds(i * 16, 16)] = comb >> 16

        def gather(j, p, sem):
            return pltpu.async_copy(hs_hbm.at[idxs_v.at[p]], rows_v.at[p], sem)

        def scatter(p):
            pltpu.sync_copy(rows_v.at[p], acc_sh.at[idxd_v.at[p]], add=True)

        unpack(0, 0)
        g0 = gather(0, 0, sem0)

        def body(jj, carry):
            j0 = jj * 2
            unpack(j0 + 1, 1)
            g1 = gather(j0 + 1, 1, sem1)
            g0 = pltpu.make_async_copy(hs_hbm.at[idxs_v.at[0]], rows_v.at[0], sem0)
            g0.wait()
            scatter(0)
            unpack(j0 + 2, 0)
            gather(j0 + 2, 0, sem0)
            g1.wait()
            scatter(1)
            return carry

        lax.fori_loop(0, K // 2 - 1, body, 0)
        unpack(K - 1, 1)
        g1 = gather(K - 1, 1, sem1)
        pltpu.make_async_copy(hs_hbm.at[idxs_v.at[0]], rows_v.at[0], sem0).wait()
        scatter(0)
        g1.wait()
        scatter(1)

        plsc.subcore_barrier()
        pltpu.sync_copy(acc_sh.at[pl.ds(r0, ROWS_PER_TILE)],
                        out_hbm.at[pl.ds(c * NPAD + r0, ROWS_PER_TILE)])

    return msg_kernel


def _spectral_norm_body(w_ref, out_ref):
    W = w_ref[...]
    din = W.shape[0]
    u = jnp.full((1, din), 1.0 / jnp.sqrt(jnp.float32(din)), dtype=jnp.float32)
    v = u
    for _ in range(3):
        v = lax.dot_general(u, W, (((1,), (0,)), ((), ())), precision=lax.Precision.HIGHEST)
        v = v / (jnp.sqrt(jnp.sum(v * v)) + 1e-12)
        u = lax.dot_general(v, W, (((1,), (1,)), ((), ())), precision=lax.Precision.HIGHEST)
        u = u / (jnp.sqrt(jnp.sum(u * u)) + 1e-12)
    wv = lax.dot_general(v, W, (((1,), (1,)), ((), ())), precision=lax.Precision.HIGHEST)
    sigma = jnp.sum(u * wv)
    out_ref[...] = W / sigma


def _hs_body(x_ref, w_ref, deg_ref, hs_ref):
    deg = jnp.sum(deg_ref[...], axis=0) + 1.0
    dis = lax.rsqrt(deg)
    h = jnp.dot(x_ref[...], w_ref[...], preferred_element_type=jnp.float32,
                precision=lax.Precision.HIGHEST)
    hs_ref[...] = h * dis[:, None]


def _final_body(a_ref, hs_ref, deg_ref, b_ref, o_ref):
    deg = jnp.sum(deg_ref[...], axis=0) + 1.0
    dis = lax.rsqrt(deg)
    acc = a_ref[0] + a_ref[1] - hs_ref[...]
    o_ref[...] = acc * dis[:, None] + b_ref[...]


def kernel(x, edge_index, W, b):
    E = edge_index.shape[1]
    src = edge_index[0].astype(jnp.int32)
    dst = edge_index[1].astype(jnp.int32)

    edges_per_blk = NC * NS * CH * 8
    EP = ((E + edges_per_blk - 1) // edges_per_blk) * edges_per_blk
    K = EP // (NC * NS * CH)
    npad_e = EP - E
    if npad_e:
        pad_rows = N_NODES + (jnp.arange(npad_e, dtype=jnp.int32) % (NPAD - N_NODES))
        srcp = jnp.concatenate([src, pad_rows])
        dstp = jnp.concatenate([dst, pad_rows])
    else:
        srcp, dstp = src, dst
    srcp = srcp.reshape(NC * NS * K, CH)
    dstp = dstp.reshape(NC * NS * K, CH)

    x_pad = jnp.pad(x, ((0, NPAD - N_NODES), (0, 0)))
    zeros_in = jnp.zeros((16 * NPP + 16,), jnp.float32)

    deg_parts = _deg_kernel_factory(K)(dstp.reshape(-1, 16), zeros_in)

    w_sn = pl.pallas_call(
        _spectral_norm_body,
        out_shape=jax.ShapeDtypeStruct(W.shape, jnp.float32),
    )(W)

    grid = NPAD // ROWS_PER_TILE
    hs = pl.pallas_call(
        _hs_body,
        grid=(grid,),
        in_specs=[
            pl.BlockSpec((ROWS_PER_TILE, D_FEAT), lambda i: (i, 0)),
            pl.BlockSpec((D_FEAT, D_HID), lambda i: (0, 0)),
            pl.BlockSpec((NC * NS, ROWS_PER_TILE), lambda i: (0, i)),
        ],
        out_specs=pl.BlockSpec((ROWS_PER_TILE, D_HID), lambda i: (i, 0)),
        out_shape=jax.ShapeDtypeStruct((NPAD, D_HID), jnp.float32),
    )(x_pad, w_sn, deg_parts)

    comb = srcp | (dstp << 16)
    acc = _msg_kernel_factory(K)(hs, comb)
    acc = acc.reshape(NC, NPAD, D_HID)

    out = pl.pallas_call(
        _final_body,
        grid=(grid,),
        in_specs=[
            pl.BlockSpec((NC, ROWS_PER_TILE, D_HID), lambda i: (0, i, 0)),
            pl.BlockSpec((ROWS_PER_TILE, D_HID), lambda i: (i, 0)),
            pl.BlockSpec((NC * NS, ROWS_PER_TILE), lambda i: (0, i)),
            pl.BlockSpec((1, D_HID), lambda i: (0, 0)),
        ],
        out_specs=pl.BlockSpec((ROWS_PER_TILE, D_HID), lambda i: (i, 0)),
        out_shape=jax.ShapeDtypeStruct((NPAD, D_HID), jnp.float32),
    )(acc, hs, deg_parts, b[None, :])

    return out[:N_NODES]


# speedup vs baseline: 35.0455x; 1.0097x over previous
import functools

import jax
import jax.numpy as jnp
from jax import lax
from jax.experimental import pallas as pl
from jax.experimental.pallas import tpu as pltpu
from jax.experimental.pallas import tpu_sc as plsc

N_NODES = 10000
D_FEAT = 128
D_HID = 128

NC = 2
NS = 16
CH = 128
NPAD = 10240
ROWS_PER_TILE = NPAD // NS

NPASS = 4
NPP = NPAD // NPASS
HR = NPP * 16 // 128
DR = NPASS * HR

_MESH = dict(core_axis_name="c", subcore_axis_name="s")


def _deg_kernel_factory(K):
    @functools.partial(
        pl.kernel,
        out_type=jax.ShapeDtypeStruct((NC * NS, NPAD), jnp.float32),
        mesh=plsc.VectorSubcoreMesh(**_MESH),
        compiler_params=pltpu.CompilerParams(needs_layout_passes=False),
        scratch_types=[
            pltpu.VMEM((K * 8, 16), jnp.int32),
            pltpu.VMEM((16 * NPP + 16,), jnp.float32),
            pltpu.VMEM((NPP,), jnp.float32),
        ],
    )
    def deg_kernel(dstp16_hbm, zeros_hbm, out_hbm, idx_v, hist_v, cnt_v):
        c = lax.axis_index("c")
        s = lax.axis_index("s")
        w = c * NS + s
        pltpu.sync_copy(dstp16_hbm.at[pl.ds(w * K * 8, K * 8)], idx_v)
        pltpu.sync_copy(zeros_hbm, hist_v)
        lane = lax.iota(jnp.int32, 16)
        ones16 = jnp.ones((16,), jnp.float32)
        zero16 = jnp.zeros((16,), jnp.float32)
        for p in range(NPASS):
            lo = p * NPP

            def body(g, carry):
                d = idx_v[g]
                m = (d >= lo) & (d < lo + NPP)
                flat = jnp.where(m, lane * NPP + (d - lo), 16 * NPP + lane)
                plsc.addupdate_scatter(hist_v, [flat], ones16)
                return carry

            lax.fori_loop(0, K * 8, body, 0, unroll=4)

            def fold(i, carry):
                acc = zero16
                for l in range(16):
                    acc = acc + hist_v[pl.ds(l * NPP + i * 16, 16)]
                    hist_v[pl.ds(l * NPP + i * 16, 16)] = zero16
                cnt_v[pl.ds(i * 16, 16)] = acc
                return carry

            lax.fori_loop(0, NPP // 16, fold, 0, unroll=2)
            pltpu.sync_copy(cnt_v, out_hbm.at[w, pl.ds(lo, NPP)])

    return deg_kernel


def _msg_kernel_factory(K):
    assert K % 2 == 0

    @functools.partial(
        pl.kernel,
        out_type=jax.ShapeDtypeStruct((NC * NPAD, D_HID), jnp.float32),
        mesh=plsc.VectorSubcoreMesh(**_MESH),
        scratch_types=[
            pltpu.VMEM((K, CH), jnp.int32),
            pltpu.VMEM((2, CH), jnp.int32),
            pltpu.VMEM((2, CH), jnp.int32),
            pltpu.VMEM((2, CH, D_HID), jnp.float32),
            pltpu.VMEM_SHARED((NPAD, D_HID), jnp.float32),
            pltpu.SemaphoreType.DMA,
            pltpu.SemaphoreType.DMA,
        ],
    )
    def msg_kernel(hs_hbm, comb_hbm, out_hbm,
                   comb_v, idxs_v, idxd_v, rows_v, acc_sh, sem0, sem1):
        c = lax.axis_index("c")
        s = lax.axis_index("s")
        r0 = s * ROWS_PER_TILE
        pltpu.sync_copy(hs_hbm.at[pl.ds(r0, ROWS_PER_TILE)],
                        acc_sh.at[pl.ds(r0, ROWS_PER_TILE)])
        base = (c * NS + s) * K
        pltpu.sync_copy(comb_hbm.at[pl.ds(base, K)], comb_v)
        plsc.subcore_barrier()

        def unpack(j, p):
            for i in range(CH // 16):
                comb = comb_v[j, pl.ds(i * 16, 16)]
                idxs_v[p, pl.ds(i * 16, 16)] = comb & 0xFFFF
                idxd_v[p, pl.ds(i * 16, 16)] = comb >> 16

        def gather(j, p, sem):
            return pltpu.async_copy(hs_hbm.at[idxs_v.at[p]], rows_v.at[p], sem)

        def scatter(p):
            pltpu.sync_copy(rows_v.at[p], acc_sh.at[idxd_v.at[p]], add=True)

        unpack(0, 0)
        g0 = gather(0, 0, sem0)

        def body(jj, carry):
            j0 = jj * 2
            unpack(j0 + 1, 1)
            g1 = gather(j0 + 1, 1, sem1)
            g0 = pltpu.make_async_copy(hs_hbm.at[idxs_v.at[0]], rows_v.at[0], sem0)
            g0.wait()
            scatter(0)
            unpack(j0 + 2, 0)
            gather(j0 + 2, 0, sem0)
            g1.wait()
            scatter(1)
            return carry

        lax.fori_loop(0, K // 2 - 1, body, 0)
        unpack(K - 1, 1)
        g1 = gather(K - 1, 1, sem1)
        pltpu.make_async_copy(hs_hbm.at[idxs_v.at[0]], rows_v.at[0], sem0).wait()
        scatter(0)
        g1.wait()
        scatter(1)

        plsc.subcore_barrier()
        pltpu.sync_copy(acc_sh.at[pl.ds(r0, ROWS_PER_TILE)],
                        out_hbm.at[pl.ds(c * NPAD + r0, ROWS_PER_TILE)])

    return msg_kernel


def _spectral_norm_body(w_ref, out_ref):
    W = w_ref[...]
    din = W.shape[0]
    u = jnp.full((1, din), 1.0 / jnp.sqrt(jnp.float32(din)), dtype=jnp.float32)
    v = u
    for _ in range(3):
        v = lax.dot_general(u, W, (((1,), (0,)), ((), ())), precision=lax.Precision.HIGHEST)
        v = v / (jnp.sqrt(jnp.sum(v * v)) + 1e-12)
        u = lax.dot_general(v, W, (((1,), (1,)), ((), ())), precision=lax.Precision.HIGHEST)
        u = u / (jnp.sqrt(jnp.sum(u * u)) + 1e-12)
    wv = lax.dot_general(v, W, (((1,), (1,)), ((), ())), precision=lax.Precision.HIGHEST)
    sigma = jnp.sum(u * wv)
    out_ref[...] = W / sigma


def _h_body(x_ref, w_ref, h_ref):
    h_ref[...] = jnp.dot(x_ref[...], w_ref[...],
                         preferred_element_type=jnp.float32,
                         precision=lax.Precision.HIGHEST)


def _scale_body(h_ref, deg_ref, hs_ref):
    deg = jnp.sum(deg_ref[...], axis=0) + 1.0
    dis = lax.rsqrt(deg)
    hs_ref[...] = h_ref[...] * dis[:, None]


def _final_body(a_ref, hs_ref, deg_ref, b_ref, o_ref):
    deg = jnp.sum(deg_ref[...], axis=0) + 1.0
    dis = lax.rsqrt(deg)
    acc = a_ref[0] + a_ref[1] - hs_ref[...]
    o_ref[...] = acc * dis[:, None] + b_ref[...]


def kernel(x, edge_index, W, b):
    E = edge_index.shape[1]
    src = edge_index[0].astype(jnp.int32)
    dst = edge_index[1].astype(jnp.int32)

    edges_per_blk = NC * NS * CH * 8
    EP = ((E + edges_per_blk - 1) // edges_per_blk) * edges_per_blk
    K = EP // (NC * NS * CH)
    npad_e = EP - E
    if npad_e:
        pad_rows = N_NODES + (jnp.arange(npad_e, dtype=jnp.int32) % (NPAD - N_NODES))
        srcp = jnp.concatenate([src, pad_rows])
        dstp = jnp.concatenate([dst, pad_rows])
    else:
        srcp, dstp = src, dst
    srcp = srcp.reshape(NC * NS * K, CH)
    dstp = dstp.reshape(NC * NS * K, CH)

    x_pad = jnp.pad(x, ((0, NPAD - N_NODES), (0, 0)))
    zeros_in = jnp.zeros((16 * NPP + 16,), jnp.float32)

    deg_parts = _deg_kernel_factory(K)(dstp.reshape(-1, 16), zeros_in)

    w_sn = pl.pallas_call(
        _spectral_norm_body,
        out_shape=jax.ShapeDtypeStruct(W.shape, jnp.float32),
    )(W)

    grid = NPAD // ROWS_PER_TILE
    h = pl.pallas_call(
        _h_body,
        grid=(grid,),
        in_specs=[
            pl.BlockSpec((ROWS_PER_TILE, D_FEAT), lambda i: (i, 0)),
            pl.BlockSpec((D_FEAT, D_HID), lambda i: (0, 0)),
        ],
        out_specs=pl.BlockSpec((ROWS_PER_TILE, D_HID), lambda i: (i, 0)),
        out_shape=jax.ShapeDtypeStruct((NPAD, D_HID), jnp.float32),
    )(x_pad, w_sn)

    hs = pl.pallas_call(
        _scale_body,
        grid=(grid,),
        in_specs=[
            pl.BlockSpec((ROWS_PER_TILE, D_HID), lambda i: (i, 0)),
            pl.BlockSpec((NC * NS, ROWS_PER_TILE), lambda i: (0, i)),
        ],
        out_specs=pl.BlockSpec((ROWS_PER_TILE, D_HID), lambda i: (i, 0)),
        out_shape=jax.ShapeDtypeStruct((NPAD, D_HID), jnp.float32),
    )(h, deg_parts)

    comb = srcp | (dstp << 16)
    acc = _msg_kernel_factory(K)(hs, comb)
    acc = acc.reshape(NC, NPAD, D_HID)

    out = pl.pallas_call(
        _final_body,
        grid=(grid,),
        in_specs=[
            pl.BlockSpec((NC, ROWS_PER_TILE, D_HID), lambda i: (0, i, 0)),
            pl.BlockSpec((ROWS_PER_TILE, D_HID), lambda i: (i, 0)),
            pl.BlockSpec((NC * NS, ROWS_PER_TILE), lambda i: (0, i)),
            pl.BlockSpec((1, D_HID), lambda i: (0, 0)),
        ],
        out_specs=pl.BlockSpec((ROWS_PER_TILE, D_HID), lambda i: (i, 0)),
        out_shape=jax.ShapeDtypeStruct((NPAD, D_HID), jnp.float32),
    )(acc, hs, deg_parts, b[None, :])

    return out[:N_NODES]


# speedup vs baseline: 35.6994x; 1.0187x over previous
import functools

import jax
import jax.numpy as jnp
from jax import lax
from jax.experimental import pallas as pl
from jax.experimental.pallas import tpu as pltpu
from jax.experimental.pallas import tpu_sc as plsc

N_NODES = 10000
D_FEAT = 128
D_HID = 128

NC = 2
NS = 16
CH = 128
NPAD = 10240
ROWS_PER_TILE = NPAD // NS

NPASS = 4
NPP = NPAD // NPASS
HR = NPP * 16 // 128
DR = NPASS * HR

_MESH = dict(core_axis_name="c", subcore_axis_name="s")


def _deg_kernel_factory(K):
    @functools.partial(
        pl.kernel,
        out_type=jax.ShapeDtypeStruct((NC * NS, NPAD), jnp.float32),
        mesh=plsc.VectorSubcoreMesh(**_MESH),
        compiler_params=pltpu.CompilerParams(needs_layout_passes=False),
        scratch_types=[
            pltpu.VMEM((K * 8, 16), jnp.int32),
            pltpu.VMEM((16 * NPP + 16,), jnp.float32),
            pltpu.VMEM((NPP,), jnp.float32),
        ],
    )
    def deg_kernel(dstp16_hbm, zeros_hbm, out_hbm, idx_v, hist_v, cnt_v):
        c = lax.axis_index("c")
        s = lax.axis_index("s")
        w = c * NS + s
        pltpu.sync_copy(dstp16_hbm.at[pl.ds(w * K * 8, K * 8)], idx_v)
        pltpu.sync_copy(zeros_hbm, hist_v)
        lane = lax.iota(jnp.int32, 16)
        ones16 = jnp.ones((16,), jnp.float32)
        zero16 = jnp.zeros((16,), jnp.float32)
        for p in range(NPASS):
            lo = p * NPP

            def body(g, carry):
                d = idx_v[g]
                m = (d >= lo) & (d < lo + NPP)
                flat = jnp.where(m, lane * NPP + (d - lo), 16 * NPP + lane)
                plsc.addupdate_scatter(hist_v, [flat], ones16)
                return carry

            lax.fori_loop(0, K * 8, body, 0, unroll=8)

            def fold(i, carry):
                acc = zero16
                for l in range(16):
                    acc = acc + hist_v[pl.ds(l * NPP + i * 16, 16)]
                    hist_v[pl.ds(l * NPP + i * 16, 16)] = zero16
                cnt_v[pl.ds(i * 16, 16)] = acc
                return carry

            lax.fori_loop(0, NPP // 16, fold, 0, unroll=4)
            pltpu.sync_copy(cnt_v, out_hbm.at[w, pl.ds(lo, NPP)])

    return deg_kernel


def _msg_kernel_factory(K):
    assert K % 2 == 0

    @functools.partial(
        pl.kernel,
        out_type=jax.ShapeDtypeStruct((NC * NPAD, D_HID), jnp.float32),
        mesh=plsc.VectorSubcoreMesh(**_MESH),
        scratch_types=[
            pltpu.VMEM((K, CH), jnp.int32),
            pltpu.VMEM((2, CH), jnp.int32),
            pltpu.VMEM((2, CH), jnp.int32),
            pltpu.VMEM((2, CH, D_HID), jnp.float32),
            pltpu.VMEM_SHARED((NPAD, D_HID), jnp.float32),
            pltpu.SemaphoreType.DMA,
            pltpu.SemaphoreType.DMA,
        ],
    )
    def msg_kernel(hs_hbm, comb_hbm, out_hbm,
                   comb_v, idxs_v, idxd_v, rows_v, acc_sh, sem0, sem1):
        c = lax.axis_index("c")
        s = lax.axis_index("s")
        r0 = s * ROWS_PER_TILE
        init = pltpu.async_copy(hs_hbm.at[pl.ds(r0, ROWS_PER_TILE)],
                                acc_sh.at[pl.ds(r0, ROWS_PER_TILE)], sem0)
        base = (c * NS + s) * K
        pltpu.sync_copy(comb_hbm.at[pl.ds(base, K)], comb_v)
        init.wait()
        plsc.subcore_barrier()

        def unpack(j, p):
            for i in range(CH // 16):
                comb = comb_v[j, pl.ds(i * 16, 16)]
                idxs_v[p, pl.ds(i * 16, 16)] = comb & 0xFFFF
                idxd_v[p, pl.ds(i * 16, 16)] = comb >> 16

        def gather(j, p, sem):
            return pltpu.async_copy(hs_hbm.at[idxs_v.at[p]], rows_v.at[p], sem)

        def scatter(p):
            pltpu.sync_copy(rows_v.at[p], acc_sh.at[idxd_v.at[p]], add=True)

        unpack(0, 0)
        g0 = gather(0, 0, sem0)

        def body(jj, carry):
            j0 = jj * 2
            unpack(j0 + 1, 1)
            g1 = gather(j0 + 1, 1, sem1)
            g0 = pltpu.make_async_copy(hs_hbm.at[idxs_v.at[0]], rows_v.at[0], sem0)
            g0.wait()
            scatter(0)
            unpack(j0 + 2, 0)
            gather(j0 + 2, 0, sem0)
            g1.wait()
            scatter(1)
            return carry

        lax.fori_loop(0, K // 2 - 1, body, 0)
        unpack(K - 1, 1)
        g1 = gather(K - 1, 1, sem1)
        pltpu.make_async_copy(hs_hbm.at[idxs_v.at[0]], rows_v.at[0], sem0).wait()
        scatter(0)
        g1.wait()
        scatter(1)

        plsc.subcore_barrier()
        pltpu.sync_copy(acc_sh.at[pl.ds(r0, ROWS_PER_TILE)],
                        out_hbm.at[pl.ds(c * NPAD + r0, ROWS_PER_TILE)])

    return msg_kernel


def _spectral_norm_body(w_ref, out_ref):
    W = w_ref[...]
    din = W.shape[0]
    u = jnp.full((1, din), 1.0 / jnp.sqrt(jnp.float32(din)), dtype=jnp.float32)
    v = u
    for _ in range(3):
        v = lax.dot_general(u, W, (((1,), (0,)), ((), ())), precision=lax.Precision.HIGHEST)
        v = v / (jnp.sqrt(jnp.sum(v * v)) + 1e-12)
        u = lax.dot_general(v, W, (((1,), (1,)), ((), ())), precision=lax.Precision.HIGHEST)
        u = u / (jnp.sqrt(jnp.sum(u * u)) + 1e-12)
    wv = lax.dot_general(v, W, (((1,), (1,)), ((), ())), precision=lax.Precision.HIGHEST)
    sigma = jnp.sum(u * wv)
    out_ref[...] = W / sigma


def _h_body(x_ref, w_ref, h_ref):
    h_ref[...] = jnp.dot(x_ref[...], w_ref[...],
                         preferred_element_type=jnp.float32,
                         precision=lax.Precision.HIGHEST)


def _scale_body(h_ref, deg_ref, hs_ref):
    deg = jnp.sum(deg_ref[...], axis=0) + 1.0
    dis = lax.rsqrt(deg)
    hs_ref[...] = h_ref[...] * dis[:, None]


def _final_body(a_ref, hs_ref, deg_ref, b_ref, o_ref):
    deg = jnp.sum(deg_ref[...], axis=0) + 1.0
    dis = lax.rsqrt(deg)
    acc = a_ref[0] + a_ref[1] - hs_ref[...]
    o_ref[...] = acc * dis[:, None] + b_ref[...]


def kernel(x, edge_index, W, b):
    E = edge_index.shape[1]
    src = edge_index[0].astype(jnp.int32)
    dst = edge_index[1].astype(jnp.int32)

    edges_per_blk = NC * NS * CH * 8
    EP = ((E + edges_per_blk - 1) // edges_per_blk) * edges_per_blk
    K = EP // (NC * NS * CH)
    npad_e = EP - E
    if npad_e:
        pad_rows = N_NODES + (jnp.arange(npad_e, dtype=jnp.int32) % (NPAD - N_NODES))
        srcp = jnp.concatenate([src, pad_rows])
        dstp = jnp.concatenate([dst, pad_rows])
    else:
        srcp, dstp = src, dst
    srcp = srcp.reshape(NC * NS * K, CH)
    dstp = dstp.reshape(NC * NS * K, CH)

    x_pad = jnp.pad(x, ((0, NPAD - N_NODES), (0, 0)))
    zeros_in = jnp.zeros((16 * NPP + 16,), jnp.float32)

    deg_parts = _deg_kernel_factory(K)(dstp.reshape(-1, 16), zeros_in)

    w_sn = pl.pallas_call(
        _spectral_norm_body,
        out_shape=jax.ShapeDtypeStruct(W.shape, jnp.float32),
    )(W)

    grid = NPAD // ROWS_PER_TILE
    h = pl.pallas_call(
        _h_body,
        grid=(grid,),
        in_specs=[
            pl.BlockSpec((ROWS_PER_TILE, D_FEAT), lambda i: (i, 0)),
            pl.BlockSpec((D_FEAT, D_HID), lambda i: (0, 0)),
        ],
        out_specs=pl.BlockSpec((ROWS_PER_TILE, D_HID), lambda i: (i, 0)),
        out_shape=jax.ShapeDtypeStruct((NPAD, D_HID), jnp.float32),
    )(x_pad, w_sn)

    hs = pl.pallas_call(
        _scale_body,
        grid=(grid,),
        in_specs=[
            pl.BlockSpec((ROWS_PER_TILE, D_HID), lambda i: (i, 0)),
            pl.BlockSpec((NC * NS, ROWS_PER_TILE), lambda i: (0, i)),
        ],
        out_specs=pl.BlockSpec((ROWS_PER_TILE, D_HID), lambda i: (i, 0)),
        out_shape=jax.ShapeDtypeStruct((NPAD, D_HID), jnp.float32),
    )(h, deg_parts)

    comb = srcp | (dstp << 16)
    acc = _msg_kernel_factory(K)(hs, comb)
    acc = acc.reshape(NC, NPAD, D_HID)

    out = pl.pallas_call(
        _final_body,
        grid=(grid,),
        in_specs=[
            pl.BlockSpec((NC, ROWS_PER_TILE, D_HID), lambda i: (0, i, 0)),
            pl.BlockSpec((ROWS_PER_TILE, D_HID), lambda i: (i, 0)),
            pl.BlockSpec((NC * NS, ROWS_PER_TILE), lambda i: (0, i)),
            pl.BlockSpec((1, D_HID), lambda i: (0, 0)),
        ],
        out_specs=pl.BlockSpec((ROWS_PER_TILE, D_HID), lambda i: (i, 0)),
        out_shape=jax.ShapeDtypeStruct((NPAD, D_HID), jnp.float32),
    )(acc, hs, deg_parts, b[None, :])

    return out[:N_NODES]
